# Initial kernel scaffold; baseline (speedup 1.0000x reference)
import jax
import jax.numpy as jnp
from jax.experimental import pallas as pl


def kernel(features, edge_index, W1, as1, ad1, b1, W2, as2, ad2, b2, Wg1, bg1, Wg2, bg2):
    raise NotImplementedError("write your pallas kernel here")


# speedup vs baseline: 12.3411x; 12.3411x over previous
import functools

import jax
import jax.numpy as jnp
from jax import lax
from jax.experimental import pallas as pl
from jax.experimental.pallas import tpu as pltpu
from jax.experimental.pallas import tpu_sc as plsc

N = 10000
NP = 10240
D = 128
E = 320000
NC, NS, LANES = 2, 16, 16
NW = NC * NS
EW = 10112
EPAD = EW * NW
CH = EW // 128
RS = NP // NS
NBLK = NP // 128

f32 = jnp.float32
i32 = jnp.int32


def _k1_body(f_ref, w1_ref, wg1_ref, as1_ref, ad1_ref,
             xp1_ref, xg1_ref, s1_ref, d1_ref):
    f = f_ref[...]
    xp = jnp.dot(f, w1_ref[...], preferred_element_type=f32)
    xp1_ref[...] = xp
    xg1_ref[...] = jnp.dot(f, wg1_ref[...], preferred_element_type=f32)
    s1_ref[...] = jnp.dot(xp, as1_ref[...], preferred_element_type=f32)
    d1_ref[...] = jnp.dot(xp, ad1_ref[...], preferred_element_type=f32)


def _k3_body(acc_ref, dnm_ref, deg_ref, b1_ref, w2_ref, as2_ref, ad2_ref,
             xg1_ref, xp2_ref, s2_ref, d2_ref, ypre1_ref, dis_ref):
    acc = acc_ref[0] + acc_ref[1]
    dnm = jnp.sum(dnm_ref[:, 0, 0, :], axis=0)
    h1 = acc * (1.0 / (dnm + 1e-16))[:, None] + b1_ref[...]
    h1 = jnp.where(h1 > 0, h1, jnp.exp(h1) - 1.0)
    xp2 = jnp.dot(h1, w2_ref[...], preferred_element_type=f32)
    xp2_ref[...] = xp2
    s2_ref[...] = jnp.dot(xp2, as2_ref[...], preferred_element_type=f32)
    d2_ref[...] = jnp.dot(xp2, ad2_ref[...], preferred_element_type=f32)
    deg = jnp.sum(deg_ref[:, 0, 0, :], axis=0)
    dis = jnp.where(deg > 0, lax.rsqrt(jnp.maximum(deg, 1e-12)), 0.0)
    dis_ref[...] = dis[:, None]
    ypre1_ref[...] = xg1_ref[...] * dis[:, None]


def _k5_body(acc2_ref, dnm2_ref, b2_ref, gcn1_ref, bg1_ref, dis_ref, wg2_ref,
             hgat_ref, ypre2_ref):
    acc2 = acc2_ref[0] + acc2_ref[1]
    dnm2 = jnp.sum(dnm2_ref[:, 0, 0, :], axis=0)
    hgat_ref[...] = acc2 * (1.0 / (dnm2 + 1e-16))[:, None] + b2_ref[...]
    dis = dis_ref[...]
    g1 = (gcn1_ref[0] + gcn1_ref[1]) * dis + bg1_ref[...]
    hg1 = jnp.maximum(g1, 0.0)
    ypre2_ref[...] = jnp.dot(hg1, wg2_ref[...], preferred_element_type=f32) * dis


def _k7_body(gcn2_ref, dis_ref, bg2_ref, hgcn_ref):
    hgcn_ref[...] = (gcn2_ref[0] + gcn2_ref[1]) * dis_ref[...] + bg2_ref[...]


def _full(shape):
    return pl.BlockSpec(shape, lambda i: tuple(0 for _ in shape))


def _rows(shape):
    n = len(shape)
    return pl.BlockSpec(shape, lambda i: (i,) + tuple(0 for _ in range(n - 1)))


def _mid(shape):
    return pl.BlockSpec(shape, lambda i: (0, i) + tuple(0 for _ in range(len(shape) - 2)))


_k1 = pl.pallas_call(
    _k1_body, grid=(NBLK,),
    in_specs=[_rows((128, D)), _full((D, D)), _full((D, D)),
              _full((D, 1)), _full((D, 1))],
    out_specs=[_rows((128, D)), _rows((128, D)), _rows((128, 1)), _rows((128, 1))],
    out_shape=[jax.ShapeDtypeStruct((NP, D), f32),
               jax.ShapeDtypeStruct((NP, D), f32),
               jax.ShapeDtypeStruct((NP, 1), f32),
               jax.ShapeDtypeStruct((NP, 1), f32)],
)

_k3 = pl.pallas_call(
    _k3_body, grid=(NBLK,),
    in_specs=[_mid((2, 128, D)), _mid((NW, 1, 1, 128)), _mid((NW, 1, 1, 128)),
              _full((1, D)), _full((D, D)), _full((D, 1)), _full((D, 1)),
              _rows((128, D))],
    out_specs=[_rows((128, D)), _rows((128, 1)), _rows((128, 1)),
               _rows((128, D)), _rows((128, 1))],
    out_shape=[jax.ShapeDtypeStruct((NP, D), f32),
               jax.ShapeDtypeStruct((NP, 1), f32),
               jax.ShapeDtypeStruct((NP, 1), f32),
               jax.ShapeDtypeStruct((NP, D), f32),
               jax.ShapeDtypeStruct((NP, 1), f32)],
)

_k5 = pl.pallas_call(
    _k5_body, grid=(NBLK,),
    in_specs=[_mid((2, 128, D)), _mid((NW, 1, 1, 128)), _full((1, D)),
              _mid((2, 128, D)), _full((1, D)), _rows((128, 1)), _full((D, D))],
    out_specs=[_rows((128, D)), _rows((128, D))],
    out_shape=[jax.ShapeDtypeStruct((NP, D), f32),
               jax.ShapeDtypeStruct((NP, D), f32)],
)

_k7 = pl.pallas_call(
    _k7_body, grid=(NBLK,),
    in_specs=[_mid((2, 128, D)), _rows((128, 1)), _full((1, D))],
    out_specs=[_rows((128, D))],
    out_shape=[jax.ShapeDtypeStruct((NP, D), f32)],
)


def _zero_vec(ref, n):
    z = jnp.zeros((LANES,), f32)

    def body(i, _):
        ref[pl.ds(pl.multiple_of(i * LANES, LANES), LANES)] = z
        return 0

    lax.fori_loop(0, n // LANES, body, 0)


def _zero_rows(rows):
    z = jnp.zeros((LANES,), f32)

    def body(i, _):
        for cb in range(D // LANES):
            rows[i, pl.ds(cb * LANES, LANES)] = z
        return 0

    lax.fori_loop(0, 128, body, 0)


def _zero_acc_slice(s, acc_sh, rows):
    for j in range(RS // 128):
        pltpu.sync_copy(rows, acc_sh.at[pl.ds(s * RS + j * 128, 128)])


def _dump_acc_slice(c, s, acc_sh, out_hbm):
    pltpu.sync_copy(acc_sh.at[pl.ds(s * RS, RS)],
                    out_hbm.at[c, pl.ds(s * RS, RS)])


def _gat_pass(wid, src_hbm, dst_hbm, xp_hbm, acc_sh, s_v, d_v, dn_l, dg_l,
              sidx, didx, w_v, rows, cvec, gsem):
    ones16 = jnp.ones((LANES,), f32)

    def chunk(i, _):
        base = pl.multiple_of(wid * EW + i * 128, 128)
        pltpu.sync_copy(src_hbm.at[pl.ds(base, 128)], sidx)
        pltpu.sync_copy(dst_hbm.at[pl.ds(base, 128)], didx)
        cp = pltpu.async_copy(xp_hbm.at[sidx], rows, gsem)
        for g in range(8):
            si = sidx[pl.ds(g * LANES, LANES)]
            di = didx[pl.ds(g * LANES, LANES)]
            t = plsc.load_gather(s_v, [si]) + plsc.load_gather(d_v, [di])
            e = jnp.where(t >= 0.0, t, 0.2 * t)
            ee = jnp.exp(e - cvec)
            w_v[pl.ds(g * LANES, LANES)] = ee
            plsc.addupdate_scatter(dn_l, [di], ee)
            if dg_l is not None:
                plsc.addupdate_scatter(dg_l, [di], ones16)
        cp.wait()

        def scale(e2, _2):
            wv = plsc.load_gather(w_v, [jnp.full((LANES,), e2, i32)])
            for cb in range(D // LANES):
                sl = pl.ds(cb * LANES, LANES)
                rows[e2, sl] = rows[e2, sl] * wv
            return 0

        lax.fori_loop(0, 128, scale, 0)
        pltpu.sync_copy(rows, acc_sh.at[didx], add=True)
        return 0

    lax.fori_loop(0, CH, chunk, 0)


def _gcn_pass(wid, src_hbm, dst_hbm, tab_hbm, acc_sh, sidx, didx, rows, gsem):
    def chunk(i, _):
        base = pl.multiple_of(wid * EW + i * 128, 128)
        pltpu.sync_copy(src_hbm.at[pl.ds(base, 128)], sidx)
        pltpu.sync_copy(dst_hbm.at[pl.ds(base, 128)], didx)
        pltpu.async_copy(tab_hbm.at[sidx], rows, gsem).wait()
        pltpu.sync_copy(rows, acc_sh.at[didx], add=True)
        return 0

    lax.fori_loop(0, CH, chunk, 0)


_mesh = plsc.VectorSubcoreMesh(core_axis_name="c", subcore_axis_name="s",
                               num_cores=NC, num_subcores=NS)


def _kdeg_body(dst_hbm, deg_hbm, dg_l, didx):
    c = lax.axis_index("c")
    s = lax.axis_index("s")
    wid = c * NS + s
    ones16 = jnp.ones((LANES,), f32)
    _zero_vec(dg_l, NP)

    def chunk(i, _):
        base = pl.multiple_of(wid * EW + i * 128, 128)
        pltpu.sync_copy(dst_hbm.at[pl.ds(base, 128)], didx)
        for g in range(8):
            di = didx[pl.ds(g * LANES, LANES)]
            plsc.addupdate_scatter(dg_l, [di], ones16)
        return 0

    lax.fori_loop(0, CH, chunk, 0)
    pltpu.sync_copy(dg_l, deg_hbm.at[wid])


def _k2_body(s1_hbm, d1_hbm, c1_hbm, src_hbm, dst_hbm, xp_hbm,
             accp_hbm, dnm_hbm,
             s_v, d_v, dn_l, sidx, didx, w_v, rows, cbuf,
             acc_sh, gsem):
    c = lax.axis_index("c")
    s = lax.axis_index("s")
    wid = c * NS + s
    pltpu.sync_copy(s1_hbm, s_v)
    pltpu.sync_copy(d1_hbm, d_v)
    pltpu.sync_copy(c1_hbm, cbuf)
    cvec = cbuf[...]
    _zero_rows(rows)
    _zero_vec(dn_l, NP)
    _zero_acc_slice(s, acc_sh, rows)
    plsc.subcore_barrier()
    _gat_pass(wid, src_hbm, dst_hbm, xp_hbm, acc_sh, s_v, d_v, dn_l, None,
              sidx, didx, w_v, rows, cvec, gsem)
    plsc.subcore_barrier()
    _dump_acc_slice(c, s, acc_sh, accp_hbm)
    pltpu.sync_copy(dn_l, dnm_hbm.at[wid])


def _k4_body(s2_hbm, d2_hbm, c2_hbm, src_hbm, dst_hbm, xp2_hbm, ypre1_hbm,
             acc2p_hbm, gcn1p_hbm, dnm2_hbm,
             s_v, d_v, dn_l, sidx, didx, w_v, rows, cbuf,
             acc_sh, gsem):
    c = lax.axis_index("c")
    s = lax.axis_index("s")
    wid = c * NS + s
    pltpu.sync_copy(s2_hbm, s_v)
    pltpu.sync_copy(d2_hbm, d_v)
    pltpu.sync_copy(c2_hbm, cbuf)
    cvec = cbuf[...]
    _zero_rows(rows)
    _zero_vec(dn_l, NP)
    _zero_acc_slice(s, acc_sh, rows)
    plsc.subcore_barrier()
    _gat_pass(wid, src_hbm, dst_hbm, xp2_hbm, acc_sh, s_v, d_v, dn_l, None,
              sidx, didx, w_v, rows, cvec, gsem)
    plsc.subcore_barrier()
    _dump_acc_slice(c, s, acc_sh, acc2p_hbm)
    pltpu.sync_copy(dn_l, dnm2_hbm.at[wid])
    plsc.subcore_barrier()
    _zero_rows(rows)
    _zero_acc_slice(s, acc_sh, rows)
    plsc.subcore_barrier()
    _gcn_pass(wid, src_hbm, dst_hbm, ypre1_hbm, acc_sh, sidx, didx, rows, gsem)
    plsc.subcore_barrier()
    _dump_acc_slice(c, s, acc_sh, gcn1p_hbm)


def _k6_body(src_hbm, dst_hbm, ypre2_hbm,
             gcn2p_hbm,
             sidx, didx, rows,
             acc_sh, gsem):
    c = lax.axis_index("c")
    s = lax.axis_index("s")
    wid = c * NS + s
    _zero_rows(rows)
    _zero_acc_slice(s, acc_sh, rows)
    plsc.subcore_barrier()
    _gcn_pass(wid, src_hbm, dst_hbm, ypre2_hbm, acc_sh, sidx, didx, rows, gsem)
    plsc.subcore_barrier()
    _dump_acc_slice(c, s, acc_sh, gcn2p_hbm)


_kdeg = pl.kernel(
    _kdeg_body,
    out_type=[jax.ShapeDtypeStruct((NW, NP), f32)],
    mesh=_mesh,
    compiler_params=pltpu.CompilerParams(needs_layout_passes=False),
    scratch_types=[
        pltpu.VMEM((NP,), f32),
        pltpu.VMEM((128,), i32),
    ],
)

_k2 = pl.kernel(
    _k2_body,
    out_type=[jax.ShapeDtypeStruct((NC, NP, D), f32),
              jax.ShapeDtypeStruct((NW, NP), f32)],
    mesh=_mesh,
    compiler_params=pltpu.CompilerParams(needs_layout_passes=False),
    scratch_types=[
        pltpu.VMEM((NP,), f32), pltpu.VMEM((NP,), f32),
        pltpu.VMEM((NP,), f32),
        pltpu.VMEM((128,), i32), pltpu.VMEM((128,), i32),
        pltpu.VMEM((128,), f32),
        pltpu.VMEM((128, D), f32),
        pltpu.VMEM((LANES,), f32),
        pltpu.VMEM_SHARED((NP, D), f32),
        pltpu.SemaphoreType.DMA,
    ],
)

_k4 = pl.kernel(
    _k4_body,
    out_type=[jax.ShapeDtypeStruct((NC, NP, D), f32),
              jax.ShapeDtypeStruct((NC, NP, D), f32),
              jax.ShapeDtypeStruct((NW, NP), f32)],
    mesh=_mesh,
    compiler_params=pltpu.CompilerParams(needs_layout_passes=False),
    scratch_types=[
        pltpu.VMEM((NP,), f32), pltpu.VMEM((NP,), f32),
        pltpu.VMEM((NP,), f32),
        pltpu.VMEM((128,), i32), pltpu.VMEM((128,), i32),
        pltpu.VMEM((128,), f32),
        pltpu.VMEM((128, D), f32),
        pltpu.VMEM((LANES,), f32),
        pltpu.VMEM_SHARED((NP, D), f32),
        pltpu.SemaphoreType.DMA,
    ],
)

_k6 = pl.kernel(
    _k6_body,
    out_type=[jax.ShapeDtypeStruct((NC, NP, D), f32)],
    mesh=_mesh,
    compiler_params=pltpu.CompilerParams(needs_layout_passes=False),
    scratch_types=[
        pltpu.VMEM((128,), i32), pltpu.VMEM((128,), i32),
        pltpu.VMEM((128, D), f32),
        pltpu.VMEM_SHARED((NP, D), f32),
        pltpu.SemaphoreType.DMA,
    ],
)


def kernel(features, edge_index, W1, as1, ad1, b1, W2, as2, ad2, b2,
           Wg1, bg1, Wg2, bg2):
    fpad = jnp.pad(features.astype(f32), ((0, NP - N), (0, 0)))
    src = edge_index[0].astype(i32)
    dst = edge_index[1].astype(i32)
    srcp = jnp.pad(src, (0, EPAD - E))
    dstp = jnp.pad(dst, (0, EPAD - E), constant_values=NP - 1)

    as1c = as1.reshape(D, 1).astype(f32)
    ad1c = ad1.reshape(D, 1).astype(f32)
    as2c = as2.reshape(D, 1).astype(f32)
    ad2c = ad2.reshape(D, 1).astype(f32)
    b1r = b1.reshape(1, D).astype(f32)
    b2r = b2.reshape(1, D).astype(f32)
    bg1r = bg1.reshape(1, D).astype(f32)
    bg2r = bg2.reshape(1, D).astype(f32)

    xp1, xg1, s1, d1 = _k1(fpad, W1.astype(f32), Wg1.astype(f32), as1c, ad1c)
    C1 = jnp.maximum(jnp.max(s1) + jnp.max(d1), 0.0)
    c1v = jnp.full((LANES,), C1, f32)

    deg, = _kdeg(dstp)
    accp, dnm = _k2(s1.reshape(NP), d1.reshape(NP), c1v, srcp, dstp, xp1)

    xp2, s2, d2, ypre1, dis = _k3(accp, dnm.reshape(NW, NBLK, 1, 128),
                                  deg.reshape(NW, NBLK, 1, 128), b1r,
                                  W2.astype(f32), as2c, ad2c, xg1)
    C2 = jnp.maximum(jnp.max(s2) + jnp.max(d2), 0.0)
    c2v = jnp.full((LANES,), C2, f32)

    acc2p, gcn1p, dnm2 = _k4(s2.reshape(NP), d2.reshape(NP), c2v,
                             srcp, dstp, xp2, ypre1)

    hgat, ypre2 = _k5(acc2p, dnm2.reshape(NW, NBLK, 1, 128), b2r, gcn1p, bg1r,
                      dis, Wg2.astype(f32))

    gcn2p, = _k6(srcp, dstp, ypre2)

    hgcn, = _k7(gcn2p, dis, bg2r)

    return jnp.concatenate([hgat[:N], hgcn[:N]], axis=1)
